# baseline (speedup 1.0000x reference)
import functools

import jax
import jax.numpy as jnp
from jax import lax
from jax.experimental import pallas as pl
from jax.experimental.pallas import tpu as pltpu
from jax.experimental.pallas import tpu_sc as plsc

_B = 1024
_CTX = 50
_EMBED = 16
_VOCAB = 100000


_NC = 2
_NS = 16
_NW = _NC * _NS
_BPW = _B // _NW
_CHUNK_B = 2
_CHUNK = _CHUNK_B * _CTX
_NCHUNK = _BPW // _CHUNK_B
_IDX_PER_W = _BPW * _CTX


def _treesum(vs):
    while len(vs) > 1:
        nxt = [vs[i] + vs[i + 1] for i in range(0, len(vs) - 1, 2)]
        if len(vs) % 2:
            nxt.append(vs[-1])
        vs = nxt
    return vs[0]


def _pool_body(idx_hbm, table_hbm, out_hbm, idx_v, rows_v, pooled_v, sem):
    wid = lax.axis_index("s") * _NC + lax.axis_index("c")
    pltpu.sync_copy(idx_hbm.at[wid], idx_v)
    copies = [
        pltpu.async_copy(
            table_hbm.at[idx_v.at[j]],
            rows_v.at[pl.ds(j * _CHUNK, _CHUNK)],
            sem,
        )
        for j in range(_NCHUNK)
    ]
    for cp in copies:
        cp.wait()

    scale = jnp.full((_EMBED,), 1.0 / _CTX, jnp.float32)

    def body(b, carry):
        base = b * _CTX
        vs = [rows_v[base + j, :] for j in range(_CTX)]
        pooled_v[b, :] = _treesum(vs) * scale
        return carry

    lax.fori_loop(0, _BPW, body, 0)
    pltpu.sync_copy(pooled_v, out_hbm.at[pl.ds(wid * _BPW, _BPW)])


def _pool(idx, table):
    mesh = plsc.VectorSubcoreMesh(core_axis_name="c", subcore_axis_name="s")
    fn = pl.kernel(
        _pool_body,
        out_type=jax.ShapeDtypeStruct((_B, _EMBED), jnp.float32),
        mesh=mesh,
        scratch_types=[
            pltpu.VMEM((_NCHUNK, _CHUNK), jnp.int32),
            pltpu.VMEM((_IDX_PER_W, _EMBED), jnp.float32),
            pltpu.VMEM((_BPW, _EMBED), jnp.float32),
            pltpu.SemaphoreType.DMA,
        ],
        compiler_params=pltpu.CompilerParams(use_tc_tiling_on_sc=False),
    )
    return fn(idx, table)


_BM = 32
_NGRID = _B // _BM
_NBUF = 3


def _proj_body(x_ref, w_ref, b_ref, o_hbm, *scr_sem):
    scrs = scr_sem[: _NBUF]
    sems = scr_sem[_NBUF:]
    i = pl.program_id(0)
    slot = lax.rem(i, _NBUF)

    block = (
        jnp.dot(x_ref[...], w_ref[...], preferred_element_type=jnp.float32)
        + b_ref[...]
    )

    for k in range(_NBUF):

        @pl.when(slot == k)
        def _(k=k):
            @pl.when(i >= _NBUF)
            def _wait_prev():
                pltpu.make_async_copy(
                    scrs[k],
                    o_hbm.at[pl.ds((i - _NBUF) * _BM, _BM), :],
                    sems[k],
                ).wait()

            scrs[k][...] = block
            pltpu.make_async_copy(
                scrs[k],
                o_hbm.at[pl.ds(i * _BM, _BM), :],
                sems[k],
            ).start()

    @pl.when(i == _NGRID - 1)
    def _drain():
        for d in range(1, _NBUF + 1):
            j = i - _NBUF + d
            k = (_NGRID - 1 + d) % _NBUF
            pltpu.make_async_copy(
                scrs[k],
                o_hbm.at[pl.ds(j * _BM, _BM), :],
                sems[k],
            ).wait()


def _project(x, W, b2d):
    return pl.pallas_call(
        _proj_body,
        grid=(_NGRID,),
        in_specs=[
            pl.BlockSpec((_BM, _EMBED), lambda i: (i, 0)),
            pl.BlockSpec((_EMBED, _VOCAB), lambda i: (0, 0)),
            pl.BlockSpec((1, _VOCAB), lambda i: (0, 0)),
        ],
        out_specs=pl.BlockSpec(memory_space=pl.ANY),
        out_shape=jax.ShapeDtypeStruct((_B, _VOCAB), jnp.float32),
        scratch_shapes=(
            [pltpu.VMEM((_BM, _VOCAB), jnp.float32) for _ in range(_NBUF)]
            + [pltpu.SemaphoreType.DMA for _ in range(_NBUF)]
        ),
        compiler_params=pltpu.CompilerParams(
            dimension_semantics=("arbitrary",),
        ),
    )(x, W, b2d)


def kernel(inputs, emb_table, W, b):
    idx = inputs.astype(jnp.int32).reshape(_NW, _NCHUNK, _CHUNK)
    pooled = _pool(idx, emb_table)
    return _project(pooled, W, b.reshape(1, _VOCAB))


# speedup vs baseline: 2.1862x; 2.1862x over previous
import functools

import jax
import jax.numpy as jnp
from jax import lax
from jax.experimental import pallas as pl
from jax.experimental.pallas import tpu as pltpu
from jax.experimental.pallas import tpu_sc as plsc

_B = 1024
_CTX = 50
_EMBED = 16
_VOCAB = 100000


_NC = 2
_NS = 16
_NW = _NC * _NS
_BPW = _B // _NW
_CHUNK_B = 2
_CHUNK = _CHUNK_B * _CTX
_NCHUNK = _BPW // _CHUNK_B
_IDX_PER_W = _BPW * _CTX


def _treesum(vs):
    while len(vs) > 1:
        nxt = [vs[i] + vs[i + 1] for i in range(0, len(vs) - 1, 2)]
        if len(vs) % 2:
            nxt.append(vs[-1])
        vs = nxt
    return vs[0]


def _pool_body(idx_hbm, table_hbm, out_hbm, idx_v, rows_v, pooled_v, sem):
    wid = lax.axis_index("s") * _NC + lax.axis_index("c")
    pltpu.sync_copy(idx_hbm.at[wid], idx_v)
    copies = [
        pltpu.async_copy(
            table_hbm.at[idx_v.at[j]],
            rows_v.at[pl.ds(j * _CHUNK, _CHUNK)],
            sem,
        )
        for j in range(_NCHUNK)
    ]
    for cp in copies:
        cp.wait()

    scale = jnp.full((_EMBED,), 1.0 / _CTX, jnp.float32)

    def body(b, carry):
        base = b * _CTX
        vs = [rows_v[base + j, :] for j in range(_CTX)]
        pooled_v[b, :] = _treesum(vs) * scale
        return carry

    lax.fori_loop(0, _BPW, body, 0)
    pltpu.sync_copy(pooled_v, out_hbm.at[pl.ds(wid * _BPW, _BPW)])


def _pool(idx, table):
    mesh = plsc.VectorSubcoreMesh(core_axis_name="c", subcore_axis_name="s")
    fn = pl.kernel(
        _pool_body,
        out_type=jax.ShapeDtypeStruct((_B, _EMBED), jnp.float32),
        mesh=mesh,
        scratch_types=[
            pltpu.VMEM((_NCHUNK, _CHUNK), jnp.int32),
            pltpu.VMEM((_IDX_PER_W, _EMBED), jnp.float32),
            pltpu.VMEM((_BPW, _EMBED), jnp.float32),
            pltpu.SemaphoreType.DMA,
        ],
        compiler_params=pltpu.CompilerParams(use_tc_tiling_on_sc=False),
    )
    return fn(idx, table)


_VT = 2048
_NGRID = (_VOCAB + _VT - 1) // _VT
_TAIL = _VOCAB - (_NGRID - 1) * _VT
_NBUF = 3


def _proj_body(x_ref, w_ref, b_ref, o_hbm, *scr_sem):
    scrs = scr_sem[: _NBUF]
    sems = scr_sem[_NBUF:]
    i = pl.program_id(0)
    slot = lax.rem(i, _NBUF)

    block = lax.dot_general(
        w_ref[...],
        x_ref[...],
        dimension_numbers=(((0,), (1,)), ((), ())),
        preferred_element_type=jnp.float32,
    ) + b_ref[...]

    for k in range(_NBUF):

        @pl.when(slot == k)
        def _(k=k):
            @pl.when(i >= _NBUF)
            def _wait_prev():
                pltpu.make_async_copy(
                    scrs[k],
                    o_hbm.at[pl.ds((i - _NBUF) * _VT, _VT)],
                    sems[k],
                ).wait()

            scrs[k][...] = block

            @pl.when(i < _NGRID - 1)
            def _fire_full():
                pltpu.make_async_copy(
                    scrs[k],
                    o_hbm.at[pl.ds(i * _VT, _VT)],
                    sems[k],
                ).start()

    @pl.when(i == _NGRID - 1)
    def _tail_and_drain():
        kl = (_NGRID - 1) % _NBUF
        pltpu.make_async_copy(
            scrs[kl].at[pl.ds(0, _TAIL)],
            o_hbm.at[pl.ds((_NGRID - 1) * _VT, _TAIL)],
            sems[kl],
        ).start()
        for d in range(1, _NBUF):
            j = _NGRID - 1 - _NBUF + d
            pltpu.make_async_copy(
                scrs[j % _NBUF],
                o_hbm.at[pl.ds(j * _VT, _VT)],
                sems[j % _NBUF],
            ).wait()
        pltpu.make_async_copy(
            scrs[kl].at[pl.ds(0, _TAIL)],
            o_hbm.at[pl.ds((_NGRID - 1) * _VT, _TAIL)],
            sems[kl],
        ).wait()


def _project(x, W, b2d):
    return pl.pallas_call(
        _proj_body,
        grid=(_NGRID,),
        in_specs=[
            pl.BlockSpec((_B, _EMBED), lambda i: (0, 0)),
            pl.BlockSpec((_EMBED, _VT), lambda i: (0, i)),
            pl.BlockSpec((_VT, 1), lambda i: (i, 0)),
        ],
        out_specs=pl.BlockSpec(memory_space=pl.ANY),
        out_shape=jax.ShapeDtypeStruct((_VOCAB, _B), jnp.float32),
        scratch_shapes=(
            [pltpu.VMEM((_VT, _B), jnp.float32) for _ in range(_NBUF)]
            + [pltpu.SemaphoreType.DMA for _ in range(_NBUF)]
        ),
        compiler_params=pltpu.CompilerParams(
            dimension_semantics=("arbitrary",),
        ),
    )(x, W, b2d)


def kernel(inputs, emb_table, W, b):
    idx = inputs.astype(jnp.int32).reshape(_NW, _NCHUNK, _CHUNK)
    pooled = _pool(idx, emb_table)
    return _project(pooled, W, b.reshape(_VOCAB, 1)).T


# speedup vs baseline: 2.7516x; 1.2586x over previous
import functools

import jax
import jax.numpy as jnp
from jax import lax
from jax.experimental import pallas as pl
from jax.experimental.pallas import tpu as pltpu
from jax.experimental.pallas import tpu_sc as plsc

_B = 1024
_CTX = 50
_EMBED = 16
_VOCAB = 100000


_NC = 2
_NS = 16
_NW = _NC * _NS
_BPW = _B // _NW
_CHUNK_B = 2
_CHUNK = _CHUNK_B * _CTX
_NCHUNK = _BPW // _CHUNK_B
_IDX_PER_W = _BPW * _CTX


def _treesum(vs):
    while len(vs) > 1:
        nxt = [vs[i] + vs[i + 1] for i in range(0, len(vs) - 1, 2)]
        if len(vs) % 2:
            nxt.append(vs[-1])
        vs = nxt
    return vs[0]


def _pool_body(idx_hbm, table_hbm, out_hbm, idx_v, rows_v, pooled_v, sem):
    wid = lax.axis_index("s") * _NC + lax.axis_index("c")
    pltpu.sync_copy(idx_hbm.at[wid], idx_v)
    copies = [
        pltpu.async_copy(
            table_hbm.at[idx_v.at[j]],
            rows_v.at[pl.ds(j * _CHUNK, _CHUNK)],
            sem,
        )
        for j in range(_NCHUNK)
    ]
    for cp in copies:
        cp.wait()

    scale = jnp.full((_EMBED,), 1.0 / _CTX, jnp.float32)

    def body(b, carry):
        base = b * _CTX
        vs = [rows_v[base + j, :] for j in range(_CTX)]
        pooled_v[b, :] = _treesum(vs) * scale
        return carry

    lax.fori_loop(0, _BPW, body, 0)
    pltpu.sync_copy(pooled_v, out_hbm.at[pl.ds(wid * _BPW, _BPW)])


def _pool(idx, table):
    mesh = plsc.VectorSubcoreMesh(core_axis_name="c", subcore_axis_name="s")
    fn = pl.kernel(
        _pool_body,
        out_type=jax.ShapeDtypeStruct((_B, _EMBED), jnp.float32),
        mesh=mesh,
        scratch_types=[
            pltpu.VMEM((_NCHUNK, _CHUNK), jnp.int32),
            pltpu.VMEM((_IDX_PER_W, _EMBED), jnp.float32),
            pltpu.VMEM((_BPW, _EMBED), jnp.float32),
            pltpu.SemaphoreType.DMA,
        ],
        compiler_params=pltpu.CompilerParams(use_tc_tiling_on_sc=False),
    )
    return fn(idx, table)


_VT = 2048
_NGRID = (_VOCAB + _VT - 1) // _VT
_TAIL = _VOCAB - (_NGRID - 1) * _VT
_NBUF = 3


def _proj_body(x_ref, w_ref, o_hbm, *scr_sem):
    scrs = scr_sem[: _NBUF]
    sems = scr_sem[_NBUF:]
    i = pl.program_id(0)
    slot = lax.rem(i, _NBUF)

    block = lax.dot_general(
        w_ref[...],
        x_ref[...],
        dimension_numbers=(((0,), (1,)), ((), ())),
        preferred_element_type=jnp.float32,
    )

    for k in range(_NBUF):

        @pl.when(slot == k)
        def _(k=k):
            @pl.when(i >= _NBUF)
            def _wait_prev():
                pltpu.make_async_copy(
                    scrs[k],
                    o_hbm.at[pl.ds((i - _NBUF) * _VT, _VT)],
                    sems[k],
                ).wait()

            scrs[k][...] = block

            @pl.when(i < _NGRID - 1)
            def _fire_full():
                pltpu.make_async_copy(
                    scrs[k],
                    o_hbm.at[pl.ds(i * _VT, _VT)],
                    sems[k],
                ).start()

    @pl.when(i == _NGRID - 1)
    def _tail_and_drain():
        kl = (_NGRID - 1) % _NBUF
        pltpu.make_async_copy(
            scrs[kl].at[pl.ds(0, _TAIL)],
            o_hbm.at[pl.ds((_NGRID - 1) * _VT, _TAIL)],
            sems[kl],
        ).start()
        for d in range(1, _NBUF):
            j = _NGRID - 1 - _NBUF + d
            pltpu.make_async_copy(
                scrs[j % _NBUF],
                o_hbm.at[pl.ds(j * _VT, _VT)],
                sems[j % _NBUF],
            ).wait()
        pltpu.make_async_copy(
            scrs[kl].at[pl.ds(0, _TAIL)],
            o_hbm.at[pl.ds((_NGRID - 1) * _VT, _TAIL)],
            sems[kl],
        ).wait()


def _project(x_aug, w_aug):
    k_aug = _EMBED + 1
    return pl.pallas_call(
        _proj_body,
        grid=(_NGRID,),
        in_specs=[
            pl.BlockSpec((_B, k_aug), lambda i: (0, 0)),
            pl.BlockSpec((k_aug, _VT), lambda i: (0, i)),
        ],
        out_specs=pl.BlockSpec(memory_space=pl.ANY),
        out_shape=jax.ShapeDtypeStruct((_VOCAB, _B), jnp.float32),
        scratch_shapes=(
            [pltpu.VMEM((_VT, _B), jnp.float32) for _ in range(_NBUF)]
            + [pltpu.SemaphoreType.DMA for _ in range(_NBUF)]
        ),
        compiler_params=pltpu.CompilerParams(
            dimension_semantics=("arbitrary",),
        ),
    )(x_aug, w_aug)


def kernel(inputs, emb_table, W, b):
    idx = inputs.astype(jnp.int32).reshape(_NW, _NCHUNK, _CHUNK)
    pooled = _pool(idx, emb_table)
    w_aug = jnp.concatenate([W, b[None, :]], axis=0)
    x_aug = jnp.concatenate(
        [pooled, jnp.ones((_B, 1), jnp.float32)], axis=1
    )
    return _project(x_aug, w_aug).T


# speedup vs baseline: 2.7643x; 1.0046x over previous
import functools

import jax
import jax.numpy as jnp
from jax import lax
from jax.experimental import pallas as pl
from jax.experimental.pallas import tpu as pltpu
from jax.experimental.pallas import tpu_sc as plsc

_B = 1024
_CTX = 50
_EMBED = 16
_VOCAB = 100000


_NC = 2
_NS = 16
_NW = _NC * _NS
_BPW = _B // _NW
_CHUNK_B = 2
_CHUNK = _CHUNK_B * _CTX
_NCHUNK = _BPW // _CHUNK_B
_IDX_PER_W = _BPW * _CTX


def _treesum(vs):
    while len(vs) > 1:
        nxt = [vs[i] + vs[i + 1] for i in range(0, len(vs) - 1, 2)]
        if len(vs) % 2:
            nxt.append(vs[-1])
        vs = nxt
    return vs[0]


def _pool_body(idx_hbm, table_hbm, out_hbm, idx_v, rows_v, pooled_v, sem):
    wid = lax.axis_index("s") * _NC + lax.axis_index("c")
    pltpu.sync_copy(idx_hbm.at[wid], idx_v)
    copies = [
        pltpu.async_copy(
            table_hbm.at[idx_v.at[j]],
            rows_v.at[pl.ds(j * _CHUNK, _CHUNK)],
            sem,
        )
        for j in range(_NCHUNK)
    ]
    for cp in copies:
        cp.wait()

    scale = jnp.full((_EMBED,), 1.0 / _CTX, jnp.float32)

    def body(b, carry):
        base = b * _CTX
        vs = [rows_v[base + j, :] for j in range(_CTX)]
        pooled_v[b, :] = _treesum(vs) * scale
        return carry

    lax.fori_loop(0, _BPW, body, 0)
    pltpu.sync_copy(pooled_v, out_hbm.at[pl.ds(wid * _BPW, _BPW)])


def _pool(idx, table):
    mesh = plsc.VectorSubcoreMesh(core_axis_name="c", subcore_axis_name="s")
    fn = pl.kernel(
        _pool_body,
        out_type=jax.ShapeDtypeStruct((_B, _EMBED), jnp.float32),
        mesh=mesh,
        scratch_types=[
            pltpu.VMEM((_NCHUNK, _CHUNK), jnp.int32),
            pltpu.VMEM((_IDX_PER_W, _EMBED), jnp.float32),
            pltpu.VMEM((_BPW, _EMBED), jnp.float32),
            pltpu.SemaphoreType.DMA,
        ],
        compiler_params=pltpu.CompilerParams(use_tc_tiling_on_sc=False),
    )
    return fn(idx, table)


_PVT = 2048
_PGRID = (_VOCAB + _PVT - 1) // _PVT


def _prep_body(t_ref, o_ref):
    o_ref[...] = jnp.concatenate(
        [t_ref[...].T, jnp.zeros((_PVT, 128 - _EMBED), jnp.float32)],
        axis=1,
    )


def _prep(tableT):
    return pl.pallas_call(
        _prep_body,
        grid=(_PGRID,),
        in_specs=[pl.BlockSpec((_EMBED, _PVT), lambda i: (0, i))],
        out_specs=pl.BlockSpec((_PVT, 128), lambda i: (i, 0)),
        out_shape=jax.ShapeDtypeStruct((_VOCAB, 128), jnp.float32),
    )(tableT)


_VT = 2048
_NGRID = (_VOCAB + _VT - 1) // _VT
_TAIL = _VOCAB - (_NGRID - 1) * _VT
_NBUF = 3


def _proj_body(x_ref, w_ref, o_hbm, *scr_sem):
    scrs = scr_sem[: _NBUF]
    sems = scr_sem[_NBUF:]
    i = pl.program_id(0)
    slot = lax.rem(i, _NBUF)

    block = lax.dot_general(
        w_ref[...],
        x_ref[...],
        dimension_numbers=(((0,), (1,)), ((), ())),
        preferred_element_type=jnp.float32,
    )

    for k in range(_NBUF):

        @pl.when(slot == k)
        def _(k=k):
            @pl.when(i >= _NBUF)
            def _wait_prev():
                pltpu.make_async_copy(
                    scrs[k],
                    o_hbm.at[pl.ds((i - _NBUF) * _VT, _VT)],
                    sems[k],
                ).wait()

            scrs[k][...] = block

            @pl.when(i < _NGRID - 1)
            def _fire_full():
                pltpu.make_async_copy(
                    scrs[k],
                    o_hbm.at[pl.ds(i * _VT, _VT)],
                    sems[k],
                ).start()

    @pl.when(i == _NGRID - 1)
    def _tail_and_drain():
        kl = (_NGRID - 1) % _NBUF
        pltpu.make_async_copy(
            scrs[kl].at[pl.ds(0, _TAIL)],
            o_hbm.at[pl.ds((_NGRID - 1) * _VT, _TAIL)],
            sems[kl],
        ).start()
        for d in range(1, _NBUF):
            j = _NGRID - 1 - _NBUF + d
            pltpu.make_async_copy(
                scrs[j % _NBUF],
                o_hbm.at[pl.ds(j * _VT, _VT)],
                sems[j % _NBUF],
            ).wait()
        pltpu.make_async_copy(
            scrs[kl].at[pl.ds(0, _TAIL)],
            o_hbm.at[pl.ds((_NGRID - 1) * _VT, _TAIL)],
            sems[kl],
        ).wait()


def _project(x_aug, w_aug):
    k_aug = _EMBED + 1
    return pl.pallas_call(
        _proj_body,
        grid=(_NGRID,),
        in_specs=[
            pl.BlockSpec((_B, k_aug), lambda i: (0, 0)),
            pl.BlockSpec((k_aug, _VT), lambda i: (0, i)),
        ],
        out_specs=pl.BlockSpec(memory_space=pl.ANY),
        out_shape=jax.ShapeDtypeStruct((_VOCAB, _B), jnp.float32),
        scratch_shapes=(
            [pltpu.VMEM((_VT, _B), jnp.float32) for _ in range(_NBUF)]
            + [pltpu.SemaphoreType.DMA for _ in range(_NBUF)]
        ),
        compiler_params=pltpu.CompilerParams(
            dimension_semantics=("arbitrary",),
            fuse_transposed_lhs_in_matmul=True,
        ),
    )(x_aug, w_aug)


def kernel(inputs, emb_table, W, b):
    idx = inputs.astype(jnp.int32).reshape(_NW, _NCHUNK, _CHUNK) * 8
    sc_table = _prep(emb_table.T).reshape(_VOCAB * 8, _EMBED)
    pooled = _pool(idx, sc_table)
    w_aug = jnp.concatenate([W, b[None, :]], axis=0)
    x_aug = jnp.concatenate(
        [pooled, jnp.ones((_B, 1), jnp.float32)], axis=1
    )
    return _project(x_aug, w_aug).T


# speedup vs baseline: 2.9785x; 1.0775x over previous
import functools

import jax
import jax.numpy as jnp
from jax import lax
from jax.experimental import pallas as pl
from jax.experimental.pallas import tpu as pltpu
from jax.experimental.pallas import tpu_sc as plsc

_B = 1024
_CTX = 50
_EMBED = 16
_VOCAB = 100000


_NC = 2
_NS = 16
_NW = _NC * _NS
_BPW = _B // _NW
_CHUNK_B = 2
_CHUNK = _CHUNK_B * _CTX
_NCHUNK = _BPW // _CHUNK_B
_IDX_PER_W = _BPW * _CTX


def _treesum(vs):
    while len(vs) > 1:
        nxt = [vs[i] + vs[i + 1] for i in range(0, len(vs) - 1, 2)]
        if len(vs) % 2:
            nxt.append(vs[-1])
        vs = nxt
    return vs[0]


def _pool_body(idx_hbm, table_hbm, out_hbm, idx_v, rows_v, pooled_v, sem):
    wid = lax.axis_index("s") * _NC + lax.axis_index("c")
    pltpu.sync_copy(idx_hbm.at[wid], idx_v)
    copies = [
        pltpu.async_copy(
            table_hbm.at[idx_v.at[j]],
            rows_v.at[pl.ds(j * _CHUNK, _CHUNK)],
            sem,
        )
        for j in range(_NCHUNK)
    ]
    for cp in copies:
        cp.wait()

    scale = jnp.full((_EMBED,), 1.0 / _CTX, jnp.float32)

    def body(b, carry):
        base = b * _CTX
        vs = [rows_v[base + j, :] for j in range(_CTX)]
        pooled_v[b, :] = _treesum(vs) * scale
        return carry

    lax.fori_loop(0, _BPW, body, 0)
    pltpu.sync_copy(pooled_v, out_hbm.at[pl.ds(wid * _BPW, _BPW)])


def _pool(idx, table):
    mesh = plsc.VectorSubcoreMesh(core_axis_name="c", subcore_axis_name="s")
    fn = pl.kernel(
        _pool_body,
        out_type=jax.ShapeDtypeStruct((_B, _EMBED), jnp.float32),
        mesh=mesh,
        scratch_types=[
            pltpu.VMEM((_NCHUNK, _CHUNK), jnp.int32),
            pltpu.VMEM((_IDX_PER_W, _EMBED), jnp.float32),
            pltpu.VMEM((_BPW, _EMBED), jnp.float32),
            pltpu.SemaphoreType.DMA,
        ],
        compiler_params=pltpu.CompilerParams(use_tc_tiling_on_sc=False),
    )
    return fn(idx, table)


_PVT = 4096
_PGRID = (_VOCAB + _PVT - 1) // _PVT
_PTAIL = _VOCAB - (_PGRID - 1) * _PVT
_PNBUF = 3


def _prep_body(t_ref, o_hbm, *scr_sem):
    scrs = scr_sem[: _PNBUF]
    sems = scr_sem[_PNBUF:]
    i = pl.program_id(0)
    slot = lax.rem(i, _PNBUF)

    block = jnp.concatenate(
        [t_ref[...].T, jnp.zeros((_PVT, 128 - _EMBED), jnp.float32)],
        axis=1,
    )

    for k in range(_PNBUF):

        @pl.when(slot == k)
        def _(k=k):
            @pl.when(i >= _PNBUF)
            def _wait_prev():
                pltpu.make_async_copy(
                    scrs[k],
                    o_hbm.at[pl.ds((i - _PNBUF) * _PVT, _PVT)],
                    sems[k],
                ).wait()

            scrs[k][...] = block

            @pl.when(i < _PGRID - 1)
            def _fire_full():
                pltpu.make_async_copy(
                    scrs[k],
                    o_hbm.at[pl.ds(i * _PVT, _PVT)],
                    sems[k],
                ).start()

    @pl.when(i == _PGRID - 1)
    def _tail_and_drain():
        kl = (_PGRID - 1) % _PNBUF
        pltpu.make_async_copy(
            scrs[kl].at[pl.ds(0, _PTAIL)],
            o_hbm.at[pl.ds((_PGRID - 1) * _PVT, _PTAIL)],
            sems[kl],
        ).start()
        for d in range(1, _PNBUF):
            j = _PGRID - 1 - _PNBUF + d
            pltpu.make_async_copy(
                scrs[j % _PNBUF],
                o_hbm.at[pl.ds(j * _PVT, _PVT)],
                sems[j % _PNBUF],
            ).wait()
        pltpu.make_async_copy(
            scrs[kl].at[pl.ds(0, _PTAIL)],
            o_hbm.at[pl.ds((_PGRID - 1) * _PVT, _PTAIL)],
            sems[kl],
        ).wait()


def _prep(tableT):
    return pl.pallas_call(
        _prep_body,
        grid=(_PGRID,),
        in_specs=[pl.BlockSpec((_EMBED, _PVT), lambda i: (0, i))],
        out_specs=pl.BlockSpec(memory_space=pl.ANY),
        out_shape=jax.ShapeDtypeStruct((_VOCAB, 128), jnp.float32),
        scratch_shapes=(
            [pltpu.VMEM((_PVT, 128), jnp.float32) for _ in range(_PNBUF)]
            + [pltpu.SemaphoreType.DMA for _ in range(_PNBUF)]
        ),
        compiler_params=pltpu.CompilerParams(
            dimension_semantics=("arbitrary",),
        ),
    )(tableT)


_VT = 2048
_NGRID = (_VOCAB + _VT - 1) // _VT
_TAIL = _VOCAB - (_NGRID - 1) * _VT
_NBUF = 3


def _proj_body(x_ref, w_ref, o_hbm, *scr_sem):
    scrs = scr_sem[: _NBUF]
    sems = scr_sem[_NBUF:]
    i = pl.program_id(0)
    slot = lax.rem(i, _NBUF)

    block = lax.dot_general(
        w_ref[...],
        x_ref[...],
        dimension_numbers=(((0,), (1,)), ((), ())),
        preferred_element_type=jnp.float32,
    )

    for k in range(_NBUF):

        @pl.when(slot == k)
        def _(k=k):
            @pl.when(i >= _NBUF)
            def _wait_prev():
                pltpu.make_async_copy(
                    scrs[k],
                    o_hbm.at[pl.ds((i - _NBUF) * _VT, _VT)],
                    sems[k],
                ).wait()

            scrs[k][...] = block

            @pl.when(i < _NGRID - 1)
            def _fire_full():
                pltpu.make_async_copy(
                    scrs[k],
                    o_hbm.at[pl.ds(i * _VT, _VT)],
                    sems[k],
                ).start()

    @pl.when(i == _NGRID - 1)
    def _tail_and_drain():
        kl = (_NGRID - 1) % _NBUF
        pltpu.make_async_copy(
            scrs[kl].at[pl.ds(0, _TAIL)],
            o_hbm.at[pl.ds((_NGRID - 1) * _VT, _TAIL)],
            sems[kl],
        ).start()
        for d in range(1, _NBUF):
            j = _NGRID - 1 - _NBUF + d
            pltpu.make_async_copy(
                scrs[j % _NBUF],
                o_hbm.at[pl.ds(j * _VT, _VT)],
                sems[j % _NBUF],
            ).wait()
        pltpu.make_async_copy(
            scrs[kl].at[pl.ds(0, _TAIL)],
            o_hbm.at[pl.ds((_NGRID - 1) * _VT, _TAIL)],
            sems[kl],
        ).wait()


def _project(x_aug, w_aug):
    k_aug = _EMBED + 1
    return pl.pallas_call(
        _proj_body,
        grid=(_NGRID,),
        in_specs=[
            pl.BlockSpec((_B, k_aug), lambda i: (0, 0)),
            pl.BlockSpec((k_aug, _VT), lambda i: (0, i)),
        ],
        out_specs=pl.BlockSpec(memory_space=pl.ANY),
        out_shape=jax.ShapeDtypeStruct((_VOCAB, _B), jnp.float32),
        scratch_shapes=(
            [pltpu.VMEM((_VT, _B), jnp.float32) for _ in range(_NBUF)]
            + [pltpu.SemaphoreType.DMA for _ in range(_NBUF)]
        ),
        compiler_params=pltpu.CompilerParams(
            dimension_semantics=("arbitrary",),
            fuse_transposed_lhs_in_matmul=True,
        ),
    )(x_aug, w_aug)


def kernel(inputs, emb_table, W, b):
    idx = inputs.astype(jnp.int32).reshape(_NW, _NCHUNK, _CHUNK) * 8
    sc_table = _prep(emb_table.T).reshape(_VOCAB * 8, _EMBED)
    pooled = _pool(idx, sc_table)
    w_aug = jnp.concatenate([W, b[None, :]], axis=0)
    x_aug = jnp.concatenate(
        [pooled, jnp.ones((_B, 1), jnp.float32)], axis=1
    )
    return _project(x_aug, w_aug).T
